# Initial kernel scaffold; baseline (speedup 1.0000x reference)
import jax
import jax.numpy as jnp
from jax.experimental import pallas as pl


def kernel(idx, tok_table, pos_table, W, b):
    raise NotImplementedError("write your pallas kernel here")


# speedup vs baseline: 1.9622x; 1.9622x over previous
import jax
import jax.numpy as jnp
from jax import lax
from jax.experimental import pallas as pl

VOCAB = 65
EMB = 32
TT = 512


def _tc_body(idx_ref, tok_ref, pos_ref, w_ref, b_ref, out_ref):
    idx_v = idx_ref[0, 0, :]
    iota_v = lax.broadcasted_iota(jnp.int32, (TT, VOCAB), 1)
    one_hot = (idx_v[:, None] == iota_v).astype(jnp.float32)
    tok_emb = jnp.dot(one_hot, tok_ref[...], preferred_element_type=jnp.float32)
    x = tok_emb + pos_ref[...]
    out_ref[0] = (
        jnp.dot(x, w_ref[...], preferred_element_type=jnp.float32)
        + b_ref[0][None, :]
    )


def kernel(idx, tok_table, pos_table, W, b):
    B, T = idx.shape
    n_t_tiles = T // TT
    grid = (B * n_t_tiles,)
    idx3 = idx.reshape(B * n_t_tiles, 1, TT)
    b2 = b.reshape(1, VOCAB)

    out = pl.pallas_call(
        _tc_body,
        grid=grid,
        in_specs=[
            pl.BlockSpec((1, 1, TT), lambda g: (g, 0, 0)),
            pl.BlockSpec((VOCAB, EMB), lambda g: (0, 0)),
            pl.BlockSpec((TT, EMB), lambda g: (g % n_t_tiles, 0)),
            pl.BlockSpec((EMB, VOCAB), lambda g: (0, 0)),
            pl.BlockSpec((1, VOCAB), lambda g: (0, 0)),
        ],
        out_specs=pl.BlockSpec((1, TT, VOCAB), lambda g: (g, 0, 0)),
        out_shape=jax.ShapeDtypeStruct((B * n_t_tiles, TT, VOCAB), jnp.float32),
    )(idx3, tok_table, pos_table, W, b2)
    return out.reshape(B, T, VOCAB)
